# Initial kernel scaffold; baseline (speedup 1.0000x reference)
import jax
import jax.numpy as jnp
from jax.experimental import pallas as pl


def kernel(x, edge_index, W1, b1, W2, b2, W3, b3):
    raise NotImplementedError("write your pallas kernel here")


# speedup vs baseline: 6.7550x; 6.7550x over previous
import functools

import jax
import jax.numpy as jnp
from jax import lax
from jax.experimental import pallas as pl
from jax.experimental.pallas import tpu as pltpu
from jax.experimental.pallas import tpu_sc as plsc

N_ = 10000
NP_ = 10240
D_ = 128
E_ = 320000
EP_ = 327680
NC_ = 2
NS_ = 16
NW_ = NC_ * NS_
K_ = 128
NCH_ = 80
RPT_ = NP_ // NS_
ZR_ = 8
PAD_DST_ = 10200
BN_ = 2000

_f32 = jnp.float32
_sc_mesh = plsc.VectorSubcoreMesh(core_axis_name="c", subcore_axis_name="s")


@functools.partial(
    pl.kernel,
    out_type=jax.ShapeDtypeStruct((NC_, NP_), _f32),
    mesh=_sc_mesh,
    scratch_types=[
        pltpu.VMEM((NCH_, K_), jnp.int32),
        pltpu.VMEM((K_,), _f32),
        pltpu.VMEM((RPT_,), _f32),
        pltpu.VMEM_SHARED((NP_,), _f32),
    ],
)
def _sc_deg(dstr_hbm, out_hbm, dstv, onesv, zbuf, acc):
    c = lax.axis_index("c")
    s = lax.axis_index("s")
    wid = c * NS_ + s

    def _fill(val, ref, n):
        def body(t, _):
            ref[pl.ds(t * 16, 16)] = jnp.full((16,), val, _f32)
            return 0
        lax.fori_loop(0, n // 16, body, 0)

    _fill(0.0, zbuf, RPT_)
    _fill(1.0, onesv, K_)

    pltpu.sync_copy(zbuf, acc.at[pl.ds(s * RPT_, RPT_)])
    plsc.subcore_barrier()

    pltpu.sync_copy(dstr_hbm.at[wid], dstv)

    def _chunk(i, _):
        pltpu.sync_copy(onesv, acc.at[dstv.at[i]], add=True)
        return 0
    lax.fori_loop(0, NCH_, _chunk, 0)

    plsc.subcore_barrier()

    pltpu.sync_copy(acc.at[pl.ds(s * RPT_, RPT_)],
                    out_hbm.at[c, pl.ds(s * RPT_, RPT_)])


@functools.partial(
    pl.kernel,
    out_type=jax.ShapeDtypeStruct((NC_, NP_, D_), _f32),
    mesh=_sc_mesh,
    scratch_types=[
        pltpu.VMEM((NCH_, K_), jnp.int32),
        pltpu.VMEM((NCH_, K_), jnp.int32),
        pltpu.VMEM((K_, D_), _f32),
        pltpu.VMEM((ZR_, D_), _f32),
        pltpu.SemaphoreType.DMA,
        pltpu.VMEM_SHARED((NP_, D_), _f32),
    ],
)
def _sc_agg(g_hbm, srcr_hbm, dstr_hbm, out_hbm, srcv, dstv, rowbuf, zbuf,
            sem, acc):
    c = lax.axis_index("c")
    s = lax.axis_index("s")
    wid = c * NS_ + s

    def _zrow(t, _):
        i = t // (D_ // 16)
        j = t % (D_ // 16)
        zbuf[i, pl.ds(j * 16, 16)] = jnp.zeros((16,), _f32)
        return 0
    lax.fori_loop(0, ZR_ * (D_ // 16), _zrow, 0)

    def _zcopy(r, _):
        pltpu.sync_copy(zbuf, acc.at[pl.ds(s * RPT_ + r * ZR_, ZR_)])
        return 0
    lax.fori_loop(0, RPT_ // ZR_, _zcopy, 0)
    plsc.subcore_barrier()

    pltpu.sync_copy(srcr_hbm.at[wid], srcv)
    pltpu.sync_copy(dstr_hbm.at[wid], dstv)

    def _chunk(i, _):
        pltpu.async_copy(g_hbm.at[srcv.at[i]], rowbuf, sem).wait()
        pltpu.sync_copy(rowbuf, acc.at[dstv.at[i]], add=True)
        return 0
    lax.fori_loop(0, NCH_, _chunk, 0)

    plsc.subcore_barrier()

    def _wb(r, _):
        pltpu.sync_copy(acc.at[pl.ds(s * RPT_ + r * 64, 64)],
                        out_hbm.at[c, pl.ds(s * RPT_ + r * 64, 64)])
        return 0
    lax.fori_loop(0, RPT_ // 64, _wb, 0)


def _mm1_body(degp_ref, x_ref, w_ref, dis_ref, selfc_ref, h_ref, g_ref):
    deg = degp_ref[:, 0] + degp_ref[:, 1] + 1.0
    dis = lax.rsqrt(deg)[:, None]
    selfc = (1.0 / deg)[:, None]
    h = jnp.dot(x_ref[...], w_ref[...], preferred_element_type=_f32,
                precision=lax.Precision.HIGHEST)
    dis_ref[...] = dis
    selfc_ref[...] = selfc
    h_ref[...] = h
    g_ref[...] = h * dis


_tc_mm1 = pl.pallas_call(
    _mm1_body,
    grid=(N_ // BN_,),
    in_specs=[
        pl.BlockSpec((BN_, NC_), lambda i: (i, 0)),
        pl.BlockSpec((BN_, D_), lambda i: (i, 0)),
        pl.BlockSpec((D_, D_), lambda i: (0, 0)),
    ],
    out_specs=[
        pl.BlockSpec((BN_, 1), lambda i: (i, 0)),
        pl.BlockSpec((BN_, 1), lambda i: (i, 0)),
        pl.BlockSpec((BN_, D_), lambda i: (i, 0)),
        pl.BlockSpec((BN_, D_), lambda i: (i, 0)),
    ],
    out_shape=[
        jax.ShapeDtypeStruct((N_, 1), _f32),
        jax.ShapeDtypeStruct((N_, 1), _f32),
        jax.ShapeDtypeStruct((N_, D_), _f32),
        jax.ShapeDtypeStruct((N_, D_), _f32),
    ],
)


def _mid_body(accp_ref, hprev_ref, dis_ref, selfc_ref, b_ref, w_ref,
              hn_ref, gn_ref):
    agg = accp_ref[0] + accp_ref[1]
    out = dis_ref[...] * agg + selfc_ref[...] * hprev_ref[...] + b_ref[...]
    out = jnp.maximum(out, 0.0)
    hn = jnp.dot(out, w_ref[...], preferred_element_type=_f32,
                 precision=lax.Precision.HIGHEST)
    hn_ref[...] = hn
    gn_ref[...] = hn * dis_ref[...]


_tc_mid = pl.pallas_call(
    _mid_body,
    grid=(N_ // BN_,),
    in_specs=[
        pl.BlockSpec((NC_, BN_, D_), lambda i: (0, i, 0)),
        pl.BlockSpec((BN_, D_), lambda i: (i, 0)),
        pl.BlockSpec((BN_, 1), lambda i: (i, 0)),
        pl.BlockSpec((BN_, 1), lambda i: (i, 0)),
        pl.BlockSpec((1, D_), lambda i: (0, 0)),
        pl.BlockSpec((D_, D_), lambda i: (0, 0)),
    ],
    out_specs=[
        pl.BlockSpec((BN_, D_), lambda i: (i, 0)),
        pl.BlockSpec((BN_, D_), lambda i: (i, 0)),
    ],
    out_shape=[
        jax.ShapeDtypeStruct((N_, D_), _f32),
        jax.ShapeDtypeStruct((N_, D_), _f32),
    ],
)


def _fin_body(accp_ref, hprev_ref, dis_ref, selfc_ref, b_ref, out_ref):
    agg = accp_ref[0] + accp_ref[1]
    out_ref[...] = (dis_ref[...] * agg + selfc_ref[...] * hprev_ref[...]
                    + b_ref[...])


_tc_fin = pl.pallas_call(
    _fin_body,
    grid=(N_ // BN_,),
    in_specs=[
        pl.BlockSpec((NC_, BN_, D_), lambda i: (0, i, 0)),
        pl.BlockSpec((BN_, D_), lambda i: (i, 0)),
        pl.BlockSpec((BN_, 1), lambda i: (i, 0)),
        pl.BlockSpec((BN_, 1), lambda i: (i, 0)),
        pl.BlockSpec((1, D_), lambda i: (0, 0)),
    ],
    out_specs=pl.BlockSpec((BN_, D_), lambda i: (i, 0)),
    out_shape=jax.ShapeDtypeStruct((N_, D_), _f32),
)


@jax.jit
def kernel(x, edge_index, W1, b1, W2, b2, W3, b3):
    ei = edge_index.astype(jnp.int32)
    npad = EP_ - E_
    srcr = jnp.concatenate(
        [ei[0], jnp.zeros((npad,), jnp.int32)]).reshape(NW_, NCH_, K_)
    dstr = jnp.concatenate(
        [ei[1], jnp.full((npad,), PAD_DST_, jnp.int32)]).reshape(NW_, NCH_, K_)

    degp = _sc_deg(dstr)
    dis, selfc, h1, g1 = _tc_mm1(degp[:, :N_].T, x, W1)

    acc1 = _sc_agg(g1, srcr, dstr)
    h2, g2 = _tc_mid(acc1, h1, dis, selfc, b1.reshape(1, D_), W2)

    acc2 = _sc_agg(g2, srcr, dstr)
    h3, g3 = _tc_mid(acc2, h2, dis, selfc, b2.reshape(1, D_), W3)

    acc3 = _sc_agg(g3, srcr, dstr)
    return _tc_fin(acc3, h3, dis, selfc, b3.reshape(1, D_))


# speedup vs baseline: 7.2029x; 1.0663x over previous
import functools

import jax
import jax.numpy as jnp
from jax import lax
from jax.experimental import pallas as pl
from jax.experimental.pallas import tpu as pltpu
from jax.experimental.pallas import tpu_sc as plsc

N_ = 10000
NP_ = 10240
D_ = 128
E_ = 320000
EP_ = 327680
NC_ = 2
NS_ = 16
NW_ = NC_ * NS_
K_ = 128
NCH_ = 80
RPT_ = NP_ // NS_
BN_ = 2000

_f32 = jnp.float32
_sc_mesh = plsc.VectorSubcoreMesh(core_axis_name="c", subcore_axis_name="s")


@functools.partial(
    pl.kernel,
    out_type=jax.ShapeDtypeStruct((NC_, NP_), _f32),
    mesh=_sc_mesh,
    scratch_types=[
        pltpu.VMEM((NCH_, K_), jnp.int32),
        pltpu.VMEM((K_,), _f32),
        pltpu.VMEM((RPT_,), _f32),
        pltpu.VMEM_SHARED((NP_,), _f32),
    ],
)
def _sc_deg(dstr_hbm, out_hbm, dstv, onesv, zbuf, acc):
    c = lax.axis_index("c")
    s = lax.axis_index("s")
    wid = c * NS_ + s

    def _fill(val, ref, n):
        def body(t, _):
            ref[pl.ds(t * 16, 16)] = jnp.full((16,), val, _f32)
            return 0
        lax.fori_loop(0, n // 16, body, 0)

    _fill(0.0, zbuf, RPT_)
    _fill(1.0, onesv, K_)

    pltpu.sync_copy(zbuf, acc.at[pl.ds(s * RPT_, RPT_)])
    plsc.subcore_barrier()

    pltpu.sync_copy(dstr_hbm.at[wid], dstv)

    def _chunk(i, _):
        pltpu.sync_copy(onesv, acc.at[dstv.at[i]], add=True)
        return 0
    lax.fori_loop(0, NCH_, _chunk, 0)

    plsc.subcore_barrier()

    pltpu.sync_copy(acc.at[pl.ds(s * RPT_, RPT_)],
                    out_hbm.at[c, pl.ds(s * RPT_, RPT_)])


GC_ = 16
NG_ = NCH_ // GC_


@functools.partial(
    pl.kernel,
    out_type=jax.ShapeDtypeStruct((NC_, NP_, D_), _f32),
    mesh=_sc_mesh,
    scratch_types=[
        pltpu.VMEM((GC_, K_), jnp.int32),
        pltpu.VMEM((GC_, K_), jnp.int32),
        pltpu.VMEM((K_, D_), _f32),
        pltpu.VMEM((K_, D_), _f32),
        pltpu.SemaphoreType.DMA,
        pltpu.SemaphoreType.DMA,
        pltpu.SemaphoreType.DMA,
        pltpu.SemaphoreType.DMA,
        pltpu.VMEM_SHARED((NP_, D_), _f32),
    ],
)
def _sc_agg(g_hbm, srcr_hbm, dstr_hbm, out_hbm, srcv, dstv, rb0, rb1,
            sg0, sg1, ss0, ss1, acc):
    c = lax.axis_index("c")
    s = lax.axis_index("s")
    wid = c * NS_ + s
    rb = (rb0, rb1)
    sg = (sg0, sg1)
    ss = (ss0, ss1)

    def _zrow(t, _):
        i = t // (D_ // 16)
        j = t % (D_ // 16)
        rb0[i, pl.ds(j * 16, 16)] = jnp.zeros((16,), _f32)
        return 0
    lax.fori_loop(0, K_ * (D_ // 16), _zrow, 0)

    def _zcopy(r, _):
        pltpu.sync_copy(rb0, acc.at[pl.ds(s * RPT_ + r * K_, K_)])
        return 0
    lax.fori_loop(0, RPT_ // K_, _zcopy, 0)
    plsc.subcore_barrier()

    def _group(g, _):
        pltpu.sync_copy(srcr_hbm.at[wid, pl.ds(g * GC_, GC_)], srcv)
        pltpu.sync_copy(dstr_hbm.at[wid, pl.ds(g * GC_, GC_)], dstv)
        gd = [None, None]
        sd = [None, None]
        gd[0] = pltpu.async_copy(g_hbm.at[srcv.at[0]], rb[0], sg[0])
        for j in range(GC_):
            b = j & 1
            gd[b].wait()
            if j + 1 < GC_:
                if j >= 1:
                    sd[1 - b].wait()
                gd[1 - b] = pltpu.async_copy(
                    g_hbm.at[srcv.at[j + 1]], rb[1 - b], sg[1 - b])
            sd[b] = pltpu.async_copy(rb[b], acc.at[dstv.at[j]], ss[b],
                                     add=True)
        sd[0].wait()
        sd[1].wait()
        return 0
    lax.fori_loop(0, NG_, _group, 0)

    plsc.subcore_barrier()

    def _wb(r, _):
        pltpu.sync_copy(acc.at[pl.ds(s * RPT_ + r * 64, 64)],
                        out_hbm.at[c, pl.ds(s * RPT_ + r * 64, 64)])
        return 0
    lax.fori_loop(0, RPT_ // 64, _wb, 0)


def _mm1_body(degp_ref, x_ref, w_ref, dis_ref, selfc_ref, h_ref, g_ref):
    deg = degp_ref[:, 0] + degp_ref[:, 1] + 1.0
    dis = lax.rsqrt(deg)[:, None]
    selfc = (1.0 / deg)[:, None]
    h = jnp.dot(x_ref[...], w_ref[...], preferred_element_type=_f32,
                precision=lax.Precision.HIGHEST)
    dis_ref[...] = dis
    selfc_ref[...] = selfc
    h_ref[...] = h
    g_ref[...] = h * dis


_tc_mm1 = pl.pallas_call(
    _mm1_body,
    grid=(N_ // BN_,),
    in_specs=[
        pl.BlockSpec((BN_, NC_), lambda i: (i, 0)),
        pl.BlockSpec((BN_, D_), lambda i: (i, 0)),
        pl.BlockSpec((D_, D_), lambda i: (0, 0)),
    ],
    out_specs=[
        pl.BlockSpec((BN_, 1), lambda i: (i, 0)),
        pl.BlockSpec((BN_, 1), lambda i: (i, 0)),
        pl.BlockSpec((BN_, D_), lambda i: (i, 0)),
        pl.BlockSpec((BN_, D_), lambda i: (i, 0)),
    ],
    out_shape=[
        jax.ShapeDtypeStruct((N_, 1), _f32),
        jax.ShapeDtypeStruct((N_, 1), _f32),
        jax.ShapeDtypeStruct((N_, D_), _f32),
        jax.ShapeDtypeStruct((N_, D_), _f32),
    ],
)


def _mid_body(accp_ref, hprev_ref, dis_ref, selfc_ref, b_ref, w_ref,
              hn_ref, gn_ref):
    agg = accp_ref[0] + accp_ref[1]
    out = dis_ref[...] * agg + selfc_ref[...] * hprev_ref[...] + b_ref[...]
    out = jnp.maximum(out, 0.0)
    hn = jnp.dot(out, w_ref[...], preferred_element_type=_f32,
                 precision=lax.Precision.HIGHEST)
    hn_ref[...] = hn
    gn_ref[...] = hn * dis_ref[...]


_tc_mid = pl.pallas_call(
    _mid_body,
    grid=(N_ // BN_,),
    in_specs=[
        pl.BlockSpec((NC_, BN_, D_), lambda i: (0, i, 0)),
        pl.BlockSpec((BN_, D_), lambda i: (i, 0)),
        pl.BlockSpec((BN_, 1), lambda i: (i, 0)),
        pl.BlockSpec((BN_, 1), lambda i: (i, 0)),
        pl.BlockSpec((1, D_), lambda i: (0, 0)),
        pl.BlockSpec((D_, D_), lambda i: (0, 0)),
    ],
    out_specs=[
        pl.BlockSpec((BN_, D_), lambda i: (i, 0)),
        pl.BlockSpec((BN_, D_), lambda i: (i, 0)),
    ],
    out_shape=[
        jax.ShapeDtypeStruct((N_, D_), _f32),
        jax.ShapeDtypeStruct((N_, D_), _f32),
    ],
)


def _fin_body(accp_ref, hprev_ref, dis_ref, selfc_ref, b_ref, out_ref):
    agg = accp_ref[0] + accp_ref[1]
    out_ref[...] = (dis_ref[...] * agg + selfc_ref[...] * hprev_ref[...]
                    + b_ref[...])


_tc_fin = pl.pallas_call(
    _fin_body,
    grid=(N_ // BN_,),
    in_specs=[
        pl.BlockSpec((NC_, BN_, D_), lambda i: (0, i, 0)),
        pl.BlockSpec((BN_, D_), lambda i: (i, 0)),
        pl.BlockSpec((BN_, 1), lambda i: (i, 0)),
        pl.BlockSpec((BN_, 1), lambda i: (i, 0)),
        pl.BlockSpec((1, D_), lambda i: (0, 0)),
    ],
    out_specs=pl.BlockSpec((BN_, D_), lambda i: (i, 0)),
    out_shape=jax.ShapeDtypeStruct((N_, D_), _f32),
)


@jax.jit
def kernel(x, edge_index, W1, b1, W2, b2, W3, b3):
    ei = edge_index.astype(jnp.int32)
    npad = EP_ - E_
    srcr = jnp.concatenate(
        [ei[0], jnp.zeros((npad,), jnp.int32)]).reshape(NW_, NCH_, K_)
    pad_dst = N_ + jnp.arange(npad, dtype=jnp.int32) % (NP_ - N_)
    dstr = jnp.concatenate([ei[1], pad_dst]).reshape(NW_, NCH_, K_)

    degp = _sc_deg(dstr)
    dis, selfc, h1, g1 = _tc_mm1(degp[:, :N_].T, x, W1)

    acc1 = _sc_agg(g1, srcr, dstr)
    h2, g2 = _tc_mid(acc1, h1, dis, selfc, b1.reshape(1, D_), W2)

    acc2 = _sc_agg(g2, srcr, dstr)
    h3, g3 = _tc_mid(acc2, h2, dis, selfc, b2.reshape(1, D_), W3)

    acc3 = _sc_agg(g3, srcr, dstr)
    return _tc_fin(acc3, h3, dis, selfc, b3.reshape(1, D_))


# speedup vs baseline: 7.4850x; 1.0392x over previous
import functools

import jax
import jax.numpy as jnp
from jax import lax
from jax.experimental import pallas as pl
from jax.experimental.pallas import tpu as pltpu
from jax.experimental.pallas import tpu_sc as plsc

N_ = 10000
NP_ = 10240
D_ = 128
E_ = 320000
EP_ = 327680
NC_ = 2
NS_ = 16
NW_ = NC_ * NS_
K_ = 128
NCH_ = 80
RPT_ = NP_ // NS_
BN_ = 2000

_f32 = jnp.float32
_sc_mesh = plsc.VectorSubcoreMesh(core_axis_name="c", subcore_axis_name="s")


@functools.partial(
    pl.kernel,
    out_type=jax.ShapeDtypeStruct((NC_, NP_), _f32),
    mesh=_sc_mesh,
    scratch_types=[
        pltpu.VMEM((NCH_, K_), jnp.int32),
        pltpu.VMEM((K_,), _f32),
        pltpu.VMEM((RPT_,), _f32),
        pltpu.VMEM_SHARED((NP_,), _f32),
    ],
)
def _sc_deg(dstr_hbm, out_hbm, dstv, onesv, zbuf, acc):
    c = lax.axis_index("c")
    s = lax.axis_index("s")
    wid = c * NS_ + s

    def _fill(val, ref, n):
        def body(t, _):
            ref[pl.ds(t * 16, 16)] = jnp.full((16,), val, _f32)
            return 0
        lax.fori_loop(0, n // 16, body, 0)

    _fill(0.0, zbuf, RPT_)
    _fill(1.0, onesv, K_)

    pltpu.sync_copy(zbuf, acc.at[pl.ds(s * RPT_, RPT_)])
    plsc.subcore_barrier()

    pltpu.sync_copy(dstr_hbm.at[wid], dstv)

    def _chunk(i, _):
        pltpu.sync_copy(onesv, acc.at[dstv.at[i]], add=True)
        return 0
    lax.fori_loop(0, NCH_, _chunk, 0)

    plsc.subcore_barrier()

    pltpu.sync_copy(acc.at[pl.ds(s * RPT_, RPT_)],
                    out_hbm.at[c, pl.ds(s * RPT_, RPT_)])


NB_ = 3
KS_ = 64
NSC_ = EP_ // NW_ // KS_


@functools.partial(
    pl.kernel,
    out_type=jax.ShapeDtypeStruct((NC_, NP_, D_), _f32),
    mesh=_sc_mesh,
    scratch_types=[
        pltpu.VMEM((NCH_, K_), jnp.int32),
        pltpu.VMEM((NCH_, K_), jnp.int32),
        pltpu.VMEM((KS_, D_), _f32),
        pltpu.VMEM((KS_, D_), _f32),
        pltpu.VMEM((KS_, D_), _f32),
        pltpu.SemaphoreType.DMA,
        pltpu.SemaphoreType.DMA,
        pltpu.SemaphoreType.DMA,
        pltpu.SemaphoreType.DMA,
        pltpu.SemaphoreType.DMA,
        pltpu.SemaphoreType.DMA,
        pltpu.VMEM_SHARED((NP_, D_), _f32),
    ],
)
def _sc_agg(g_hbm, srcr_hbm, dstr_hbm, out_hbm, srcv, dstv,
            rb0, rb1, rb2, sg0, sg1, sg2, ss0, ss1, ss2, acc):
    c = lax.axis_index("c")
    s = lax.axis_index("s")
    wid = c * NS_ + s
    rb = (rb0, rb1, rb2)
    sg = (sg0, sg1, sg2)
    ss = (ss0, ss1, ss2)

    def _zrow(t, _):
        i = t // (D_ // 16)
        j = t % (D_ // 16)
        rb0[i, pl.ds(j * 16, 16)] = jnp.zeros((16,), _f32)
        return 0
    lax.fori_loop(0, KS_ * (D_ // 16), _zrow, 0)

    def _zcopy(r, _):
        pltpu.sync_copy(rb0, acc.at[pl.ds(s * RPT_ + r * KS_, KS_)])
        return 0
    lax.fori_loop(0, RPT_ // KS_, _zcopy, 0)
    plsc.subcore_barrier()

    pltpu.sync_copy(srcr_hbm.at[wid], srcv)
    pltpu.sync_copy(dstr_hbm.at[wid], dstv)

    gd = [None] * NB_
    sd = [None] * NB_
    LAG = NB_ - 1
    for j in range(NSC_ + LAG):
        if j < NSC_:
            b = j % NB_
            if j >= NB_:
                sd[b].wait()
            gd[b] = pltpu.async_copy(
                g_hbm.at[srcv.at[j // 2, pl.ds((j % 2) * KS_, KS_)]],
                rb[b], sg[b])
        if j >= LAG:
            i = j - LAG
            b = i % NB_
            gd[b].wait()
            sd[b] = pltpu.async_copy(
                rb[b], acc.at[dstv.at[i // 2, pl.ds((i % 2) * KS_, KS_)]],
                ss[b], add=True)
    for i in range(NSC_ - NB_, NSC_):
        sd[i % NB_].wait()

    plsc.subcore_barrier()

    def _wb(r, _):
        pltpu.sync_copy(acc.at[pl.ds(s * RPT_ + r * 64, 64)],
                        out_hbm.at[c, pl.ds(s * RPT_ + r * 64, 64)])
        return 0
    lax.fori_loop(0, RPT_ // 64, _wb, 0)


def _mm1_body(degp_ref, x_ref, w_ref, dis_ref, selfc_ref, h_ref, g_ref):
    deg = degp_ref[:, 0] + degp_ref[:, 1] + 1.0
    dis = lax.rsqrt(deg)[:, None]
    selfc = (1.0 / deg)[:, None]
    h = jnp.dot(x_ref[...], w_ref[...], preferred_element_type=_f32,
                precision=lax.Precision.HIGHEST)
    dis_ref[...] = dis
    selfc_ref[...] = selfc
    h_ref[...] = h
    g_ref[...] = h * dis


_tc_mm1 = pl.pallas_call(
    _mm1_body,
    grid=(N_ // BN_,),
    in_specs=[
        pl.BlockSpec((BN_, NC_), lambda i: (i, 0)),
        pl.BlockSpec((BN_, D_), lambda i: (i, 0)),
        pl.BlockSpec((D_, D_), lambda i: (0, 0)),
    ],
    out_specs=[
        pl.BlockSpec((BN_, 1), lambda i: (i, 0)),
        pl.BlockSpec((BN_, 1), lambda i: (i, 0)),
        pl.BlockSpec((BN_, D_), lambda i: (i, 0)),
        pl.BlockSpec((BN_, D_), lambda i: (i, 0)),
    ],
    out_shape=[
        jax.ShapeDtypeStruct((N_, 1), _f32),
        jax.ShapeDtypeStruct((N_, 1), _f32),
        jax.ShapeDtypeStruct((N_, D_), _f32),
        jax.ShapeDtypeStruct((N_, D_), _f32),
    ],
)


def _mid_body(accp_ref, hprev_ref, dis_ref, selfc_ref, b_ref, w_ref,
              hn_ref, gn_ref):
    agg = accp_ref[0] + accp_ref[1]
    out = dis_ref[...] * agg + selfc_ref[...] * hprev_ref[...] + b_ref[...]
    out = jnp.maximum(out, 0.0)
    hn = jnp.dot(out, w_ref[...], preferred_element_type=_f32,
                 precision=lax.Precision.HIGHEST)
    hn_ref[...] = hn
    gn_ref[...] = hn * dis_ref[...]


_tc_mid = pl.pallas_call(
    _mid_body,
    grid=(N_ // BN_,),
    in_specs=[
        pl.BlockSpec((NC_, BN_, D_), lambda i: (0, i, 0)),
        pl.BlockSpec((BN_, D_), lambda i: (i, 0)),
        pl.BlockSpec((BN_, 1), lambda i: (i, 0)),
        pl.BlockSpec((BN_, 1), lambda i: (i, 0)),
        pl.BlockSpec((1, D_), lambda i: (0, 0)),
        pl.BlockSpec((D_, D_), lambda i: (0, 0)),
    ],
    out_specs=[
        pl.BlockSpec((BN_, D_), lambda i: (i, 0)),
        pl.BlockSpec((BN_, D_), lambda i: (i, 0)),
    ],
    out_shape=[
        jax.ShapeDtypeStruct((N_, D_), _f32),
        jax.ShapeDtypeStruct((N_, D_), _f32),
    ],
)


def _fin_body(accp_ref, hprev_ref, dis_ref, selfc_ref, b_ref, out_ref):
    agg = accp_ref[0] + accp_ref[1]
    out_ref[...] = (dis_ref[...] * agg + selfc_ref[...] * hprev_ref[...]
                    + b_ref[...])


_tc_fin = pl.pallas_call(
    _fin_body,
    grid=(N_ // BN_,),
    in_specs=[
        pl.BlockSpec((NC_, BN_, D_), lambda i: (0, i, 0)),
        pl.BlockSpec((BN_, D_), lambda i: (i, 0)),
        pl.BlockSpec((BN_, 1), lambda i: (i, 0)),
        pl.BlockSpec((BN_, 1), lambda i: (i, 0)),
        pl.BlockSpec((1, D_), lambda i: (0, 0)),
    ],
    out_specs=pl.BlockSpec((BN_, D_), lambda i: (i, 0)),
    out_shape=jax.ShapeDtypeStruct((N_, D_), _f32),
)


@jax.jit
def kernel(x, edge_index, W1, b1, W2, b2, W3, b3):
    ei = edge_index.astype(jnp.int32)
    npad = EP_ - E_
    srcr = jnp.concatenate(
        [ei[0], jnp.zeros((npad,), jnp.int32)]).reshape(NW_, NCH_, K_)
    pad_dst = N_ + jnp.arange(npad, dtype=jnp.int32) % (NP_ - N_)
    dstr = jnp.concatenate([ei[1], pad_dst]).reshape(NW_, NCH_, K_)

    degp = _sc_deg(dstr)
    dis, selfc, h1, g1 = _tc_mm1(degp[:, :N_].T, x, W1)

    acc1 = _sc_agg(g1, srcr, dstr)
    h2, g2 = _tc_mid(acc1, h1, dis, selfc, b1.reshape(1, D_), W2)

    acc2 = _sc_agg(g2, srcr, dstr)
    h3, g3 = _tc_mid(acc2, h2, dis, selfc, b2.reshape(1, D_), W3)

    acc3 = _sc_agg(g3, srcr, dstr)
    return _tc_fin(acc3, h3, dis, selfc, b3.reshape(1, D_))


# speedup vs baseline: 7.5119x; 1.0036x over previous
import functools

import jax
import jax.numpy as jnp
from jax import lax
from jax.experimental import pallas as pl
from jax.experimental.pallas import tpu as pltpu
from jax.experimental.pallas import tpu_sc as plsc

N_ = 10000
NP_ = 10240
D_ = 128
E_ = 320000
EP_ = 327680
NC_ = 2
NS_ = 16
NW_ = NC_ * NS_
K_ = 128
NCH_ = 80
RPT_ = NP_ // NS_
BN_ = 2000

_f32 = jnp.float32
_sc_mesh = plsc.VectorSubcoreMesh(core_axis_name="c", subcore_axis_name="s")


@functools.partial(
    pl.kernel,
    out_type=jax.ShapeDtypeStruct((NC_, NP_), _f32),
    mesh=_sc_mesh,
    scratch_types=[
        pltpu.VMEM((NCH_, K_), jnp.int32),
        pltpu.VMEM((K_,), _f32),
        pltpu.VMEM((RPT_,), _f32),
        pltpu.VMEM_SHARED((NP_,), _f32),
    ],
)
def _sc_deg(dstr_hbm, out_hbm, dstv, onesv, zbuf, acc):
    c = lax.axis_index("c")
    s = lax.axis_index("s")
    wid = c * NS_ + s

    def _fill(val, ref, n):
        def body(t, _):
            ref[pl.ds(t * 16, 16)] = jnp.full((16,), val, _f32)
            return 0
        lax.fori_loop(0, n // 16, body, 0)

    _fill(0.0, zbuf, RPT_)
    _fill(1.0, onesv, K_)

    pltpu.sync_copy(zbuf, acc.at[pl.ds(s * RPT_, RPT_)])
    plsc.subcore_barrier()

    pltpu.sync_copy(dstr_hbm.at[wid], dstv)

    def _chunk(i, _):
        pltpu.sync_copy(onesv, acc.at[dstv.at[i]], add=True)
        return 0
    lax.fori_loop(0, NCH_, _chunk, 0)

    plsc.subcore_barrier()

    pltpu.sync_copy(acc.at[pl.ds(s * RPT_, RPT_)],
                    out_hbm.at[c, pl.ds(s * RPT_, RPT_)])


NB_ = 3
KS_ = 64
NSC_ = EP_ // NW_ // KS_


@functools.partial(
    pl.kernel,
    out_type=jax.ShapeDtypeStruct((NC_, NP_, D_), _f32),
    mesh=_sc_mesh,
    scratch_types=[
        pltpu.VMEM((NCH_, K_), jnp.int32),
        pltpu.VMEM((NCH_, K_), jnp.int32),
        pltpu.VMEM((KS_, D_), _f32),
        pltpu.VMEM((KS_, D_), _f32),
        pltpu.VMEM((KS_, D_), _f32),
        pltpu.SemaphoreType.DMA,
        pltpu.SemaphoreType.DMA,
        pltpu.SemaphoreType.DMA,
        pltpu.SemaphoreType.DMA,
        pltpu.SemaphoreType.DMA,
        pltpu.SemaphoreType.DMA,
        pltpu.VMEM_SHARED((NP_, D_), _f32),
    ],
)
def _sc_agg(g_hbm, srcr_hbm, dstr_hbm, out_hbm, srcv, dstv,
            rb0, rb1, rb2, sg0, sg1, sg2, ss0, ss1, ss2, acc):
    c = lax.axis_index("c")
    s = lax.axis_index("s")
    wid = c * NS_ + s
    rb = (rb0, rb1, rb2)
    sg = (sg0, sg1, sg2)
    ss = (ss0, ss1, ss2)

    pltpu.sync_copy(srcr_hbm.at[wid], srcv)
    pltpu.sync_copy(dstr_hbm.at[wid], dstv)

    def _zrow(t, _):
        i = t // (D_ // 16)
        j = t % (D_ // 16)
        rb0[i, pl.ds(j * 16, 16)] = jnp.zeros((16,), _f32)
        return 0
    lax.fori_loop(0, KS_ * (D_ // 16), _zrow, 0)

    gd = [None] * NB_
    sd = [None] * NB_
    gd[1] = pltpu.async_copy(g_hbm.at[srcv.at[0, pl.ds(0, KS_)]],
                             rb[1], sg[1])
    gd[2] = pltpu.async_copy(g_hbm.at[srcv.at[0, pl.ds(KS_, KS_)]],
                             rb[2], sg[2])

    def _zcopy(r, _):
        pltpu.sync_copy(rb0, acc.at[pl.ds(s * RPT_ + r * KS_, KS_)])
        return 0
    lax.fori_loop(0, RPT_ // KS_, _zcopy, 0)
    plsc.subcore_barrier()

    LAG = NB_ - 1
    for j in range(NSC_ + LAG):
        if j >= 2 and j < NSC_:
            b = (j + 1) % NB_
            if j >= NB_:
                sd[b].wait()
            gd[b] = pltpu.async_copy(
                g_hbm.at[srcv.at[j // 2, pl.ds((j % 2) * KS_, KS_)]],
                rb[b], sg[b])
        if j >= LAG:
            i = j - LAG
            b = (i + 1) % NB_
            gd[b].wait()
            sd[b] = pltpu.async_copy(
                rb[b], acc.at[dstv.at[i // 2, pl.ds((i % 2) * KS_, KS_)]],
                ss[b], add=True)
    for i in range(NSC_ - NB_, NSC_):
        sd[(i + 1) % NB_].wait()

    plsc.subcore_barrier()

    NWB_ = RPT_ // 64
    for r in range(NWB_):
        pltpu.async_copy(acc.at[pl.ds(s * RPT_ + r * 64, 64)],
                         out_hbm.at[c, pl.ds(s * RPT_ + r * 64, 64)], ss0)
    for r in range(NWB_):
        pltpu.make_async_copy(acc.at[pl.ds(s * RPT_ + r * 64, 64)],
                              out_hbm.at[c, pl.ds(s * RPT_ + r * 64, 64)],
                              ss0).wait()


def _mm1_body(degp_ref, x_ref, w_ref, dis_ref, selfc_ref, h_ref, g_ref):
    deg = degp_ref[:, 0] + degp_ref[:, 1] + 1.0
    dis = lax.rsqrt(deg)[:, None]
    selfc = (1.0 / deg)[:, None]
    h = jnp.dot(x_ref[...], w_ref[...], preferred_element_type=_f32,
                precision=lax.Precision.HIGHEST)
    dis_ref[...] = dis
    selfc_ref[...] = selfc
    h_ref[...] = h
    g_ref[...] = h * dis


_tc_mm1 = pl.pallas_call(
    _mm1_body,
    grid=(N_ // BN_,),
    in_specs=[
        pl.BlockSpec((BN_, NC_), lambda i: (i, 0)),
        pl.BlockSpec((BN_, D_), lambda i: (i, 0)),
        pl.BlockSpec((D_, D_), lambda i: (0, 0)),
    ],
    out_specs=[
        pl.BlockSpec((BN_, 1), lambda i: (i, 0)),
        pl.BlockSpec((BN_, 1), lambda i: (i, 0)),
        pl.BlockSpec((BN_, D_), lambda i: (i, 0)),
        pl.BlockSpec((BN_, D_), lambda i: (i, 0)),
    ],
    out_shape=[
        jax.ShapeDtypeStruct((N_, 1), _f32),
        jax.ShapeDtypeStruct((N_, 1), _f32),
        jax.ShapeDtypeStruct((N_, D_), _f32),
        jax.ShapeDtypeStruct((N_, D_), _f32),
    ],
)


def _mid_body(accp_ref, hprev_ref, dis_ref, selfc_ref, b_ref, w_ref,
              hn_ref, gn_ref):
    agg = accp_ref[0] + accp_ref[1]
    out = dis_ref[...] * agg + selfc_ref[...] * hprev_ref[...] + b_ref[...]
    out = jnp.maximum(out, 0.0)
    hn = jnp.dot(out, w_ref[...], preferred_element_type=_f32,
                 precision=lax.Precision.HIGHEST)
    hn_ref[...] = hn
    gn_ref[...] = hn * dis_ref[...]


_tc_mid = pl.pallas_call(
    _mid_body,
    grid=(N_ // BN_,),
    in_specs=[
        pl.BlockSpec((NC_, BN_, D_), lambda i: (0, i, 0)),
        pl.BlockSpec((BN_, D_), lambda i: (i, 0)),
        pl.BlockSpec((BN_, 1), lambda i: (i, 0)),
        pl.BlockSpec((BN_, 1), lambda i: (i, 0)),
        pl.BlockSpec((1, D_), lambda i: (0, 0)),
        pl.BlockSpec((D_, D_), lambda i: (0, 0)),
    ],
    out_specs=[
        pl.BlockSpec((BN_, D_), lambda i: (i, 0)),
        pl.BlockSpec((BN_, D_), lambda i: (i, 0)),
    ],
    out_shape=[
        jax.ShapeDtypeStruct((N_, D_), _f32),
        jax.ShapeDtypeStruct((N_, D_), _f32),
    ],
)


def _fin_body(accp_ref, hprev_ref, dis_ref, selfc_ref, b_ref, out_ref):
    agg = accp_ref[0] + accp_ref[1]
    out_ref[...] = (dis_ref[...] * agg + selfc_ref[...] * hprev_ref[...]
                    + b_ref[...])


_tc_fin = pl.pallas_call(
    _fin_body,
    grid=(N_ // BN_,),
    in_specs=[
        pl.BlockSpec((NC_, BN_, D_), lambda i: (0, i, 0)),
        pl.BlockSpec((BN_, D_), lambda i: (i, 0)),
        pl.BlockSpec((BN_, 1), lambda i: (i, 0)),
        pl.BlockSpec((BN_, 1), lambda i: (i, 0)),
        pl.BlockSpec((1, D_), lambda i: (0, 0)),
    ],
    out_specs=pl.BlockSpec((BN_, D_), lambda i: (i, 0)),
    out_shape=jax.ShapeDtypeStruct((N_, D_), _f32),
)


@jax.jit
def kernel(x, edge_index, W1, b1, W2, b2, W3, b3):
    ei = edge_index.astype(jnp.int32)
    npad = EP_ - E_
    srcr = jnp.concatenate(
        [ei[0], jnp.zeros((npad,), jnp.int32)]).reshape(NW_, NCH_, K_)
    pad_dst = N_ + jnp.arange(npad, dtype=jnp.int32) % (NP_ - N_)
    dstr = jnp.concatenate([ei[1], pad_dst]).reshape(NW_, NCH_, K_)

    degp = _sc_deg(dstr)
    dis, selfc, h1, g1 = _tc_mm1(degp[:, :N_].T, x, W1)

    acc1 = _sc_agg(g1, srcr, dstr)
    h2, g2 = _tc_mid(acc1, h1, dis, selfc, b1.reshape(1, D_), W2)

    acc2 = _sc_agg(g2, srcr, dstr)
    h3, g3 = _tc_mid(acc2, h2, dis, selfc, b2.reshape(1, D_), W3)

    acc3 = _sc_agg(g3, srcr, dstr)
    return _tc_fin(acc3, h3, dis, selfc, b3.reshape(1, D_))
